# baseline (speedup 1.0000x reference)
import jax
import jax.numpy as jnp
from jax import lax
from jax.experimental import pallas as pl
from jax.experimental.pallas import tpu as pltpu
from jax.experimental.pallas import tpu_sc as plsc

N = 10000
E = 320000
K = 5
KK = K * K
DF = 128
H1 = 32
H2 = 64
G = 64
NCLS = 10

CH = 80
NW = 32
CPW = E // (NW * CH)
NRD = 625

_HIGH = jax.lax.Precision.HIGHEST
_SC_PARAMS = pltpu.CompilerParams(use_tc_tiling_on_sc=False)


def _dot(a, b):
    return jnp.dot(a, b, preferred_element_type=jnp.float32, precision=_HIGH)


def _make_edge_pass(width, with_deg):
    mesh = plsc.VectorSubcoreMesh(core_axis_name="c", subcore_axis_name="s")
    outs = [jax.ShapeDtypeStruct((2, N, width), jnp.float32)]
    if with_deg:
        outs.append(jax.ShapeDtypeStruct((2, N, 16), jnp.float32))
    scratch = [
        pltpu.VMEM((2 * CH,), jnp.int32),
        pltpu.VMEM((2, CH), jnp.int32),
        pltpu.VMEM((2 * CH, 16), jnp.float32),
        pltpu.VMEM((2 * CH, 16), jnp.float32),
        pltpu.VMEM((2, CH), jnp.int32),
        pltpu.VMEM((2, CH), jnp.int32),
        pltpu.VMEM((2, CH), jnp.int32),
        pltpu.VMEM((2, CH), jnp.int32),
        pltpu.VMEM((8 * CH, width), jnp.float32),
        pltpu.VMEM((2 * CH, width), jnp.float32),
        pltpu.VMEM((2, CH), jnp.int32),
        pltpu.VMEM_SHARED((N, width), jnp.float32),
        pltpu.SemaphoreType.DMA,
        pltpu.SemaphoreType.DMA,
        pltpu.SemaphoreType.DMA,
    ]
    if with_deg:
        scratch += [
            pltpu.VMEM((CH, 16), jnp.float32),
            pltpu.VMEM((NRD, 16), jnp.float32),
            pltpu.VMEM_SHARED((N, 16), jnp.float32),
        ]

    def body(*refs):
        if with_deg:
            (fl_h, dst_h, f0_h, f1_h, tab_h, zv_h, zd_h, ones_h,
             out_h, deg_h,
             fl_v, dst_v, f0_v, f1_v, i0_v, i1_v, i2_v, i3_v, rows_v, msg_v,
             dsh_v, acc, sem_i, sem_g, sem_s, ones_v, dbuf_v, dacc) = refs
        else:
            (fl_h, dst_h, f0_h, f1_h, tab_h, zv_h,
             out_h,
             fl_v, dst_v, f0_v, f1_v, i0_v, i1_v, i2_v, i3_v, rows_v, msg_v,
             dsh_v, acc, sem_i, sem_g, sem_s) = refs

        cid = lax.axis_index("c")
        sid = lax.axis_index("s")
        w = cid * 16 + sid
        ebase = w * CPW * CH

        def in_copies(j, b):
            e0 = ebase + j * CH
            return [
                pltpu.make_async_copy(fl_h.at[pl.ds(e0, CH)],
                                      fl_v.at[pl.ds(b * CH, CH)], sem_i),
                pltpu.make_async_copy(dst_h.at[pl.ds(e0, CH)],
                                      dst_v.at[b], sem_i),
                pltpu.make_async_copy(f0_h.at[pl.ds(e0, CH)],
                                      f0_v.at[pl.ds(b * CH, CH)], sem_i),
                pltpu.make_async_copy(f1_h.at[pl.ds(e0, CH)],
                                      f1_v.at[pl.ds(b * CH, CH)], sem_i),
            ]

        def gather_copies(b):
            return [
                pltpu.make_async_copy(tab_h.at[iv.at[b]],
                                      rows_v.at[pl.ds((4 * b + c) * CH, CH)],
                                      sem_g)
                for c, iv in enumerate((i0_v, i1_v, i2_v, i3_v))
            ]

        def scatter_copies(b):
            cps = [pltpu.make_async_copy(
                msg_v.at[pl.ds(b * CH, CH)], acc.at[dsh_v.at[b]], sem_s)]
            if with_deg:
                cps.append(pltpu.make_async_copy(
                    ones_v, dacc.at[dsh_v.at[b]], sem_s))
            return cps

        def shadow_dst(b):
            def grp(m, c2):
                dsh_v[b, pl.ds(m * 16, 16)] = dst_v[b, pl.ds(m * 16, 16)]
                return c2
            lax.fori_loop(0, CH // 16, grp, 0)

        def comp_idx(b):
            def grp(m, c2):
                o = b * CH + m * 16
                fl = fl_v[pl.ds(o, 16)]
                i0_v[b, pl.ds(m * 16, 16)] = fl
                i1_v[b, pl.ds(m * 16, 16)] = fl + 1
                i2_v[b, pl.ds(m * 16, 16)] = fl + K
                i3_v[b, pl.ds(m * 16, 16)] = fl + (K + 1)
                return c2
            lax.fori_loop(0, CH // 16, grp, 0)

        def weight(b):
            rb = 4 * b * CH

            def edge(i2, c2):
                for u in range(2):
                    i = i2 * 2 + u
                    f0 = f0_v[b * CH + i, pl.ds(0, 16)]
                    f1 = f1_v[b * CH + i, pl.ds(0, 16)]
                    for kk in range(width // 16):
                        o = kk * 16
                        r00 = rows_v[rb + i, pl.ds(o, 16)]
                        r01 = rows_v[rb + CH + i, pl.ds(o, 16)]
                        r10 = rows_v[rb + 2 * CH + i, pl.ds(o, 16)]
                        r11 = rows_v[rb + 3 * CH + i, pl.ds(o, 16)]
                        a = r00 + f1 * (r01 - r00)
                        bq = r10 + f1 * (r11 - r10)
                        msg_v[b * CH + i, pl.ds(o, 16)] = a + f0 * (bq - a)
                return c2

            lax.fori_loop(0, CH // 2, edge, 0)

        bounce = rows_v.at[pl.ds(0, NRD)]
        pltpu.sync_copy(zv_h, bounce)
        pltpu.sync_copy(bounce, acc.at[pl.ds(sid * NRD, NRD)])
        if with_deg:
            pltpu.sync_copy(zd_h, dbuf_v)
            pltpu.sync_copy(dbuf_v, dacc.at[pl.ds(sid * NRD, NRD)])
            pltpu.sync_copy(ones_h, ones_v)
        plsc.subcore_barrier()

        for cp in in_copies(0, 0):
            cp.start()
        for cp in in_copies(0, 0):
            cp.wait()
        comp_idx(0)
        for cp in gather_copies(0):
            cp.start()

        def chunk(j, carry):
            b = lax.rem(j, 2)
            nb = 1 - b

            @pl.when(j > 0)
            def _():
                for cp in scatter_copies(nb):
                    cp.wait()

            @pl.when(j + 1 < CPW)
            def _():
                for cp in in_copies(j + 1, nb):
                    cp.start()

            for cp in gather_copies(b):
                cp.wait()

            @pl.when(j + 1 < CPW)
            def _():
                for cp in in_copies(j + 1, nb):
                    cp.wait()
                comp_idx(nb)
                for cp in gather_copies(nb):
                    cp.start()

            weight(b)
            shadow_dst(b)

            for cp in scatter_copies(b):
                cp.start()
            return carry

        lax.fori_loop(0, CPW, chunk, 0)
        for cp in scatter_copies((CPW - 1) % 2):
            cp.wait()
        plsc.subcore_barrier()

        pltpu.sync_copy(acc.at[pl.ds(sid * NRD, NRD)], bounce)
        pltpu.sync_copy(bounce, out_h.at[cid, pl.ds(sid * NRD, NRD)])
        if with_deg:
            pltpu.sync_copy(dacc.at[pl.ds(sid * NRD, NRD)], dbuf_v)
            pltpu.sync_copy(dbuf_v, deg_h.at[cid, pl.ds(sid * NRD, NRD)])

    return pl.kernel(body, out_type=outs, mesh=mesh, scratch_types=scratch,
                     compiler_params=_SC_PARAMS)


_EB = 2500
_RB = 1000
_NB = N // _RB


def _edge_prep(src2, a02, a12):
    def body(s_ref, a0_ref, a1_ref, fl_ref, f0_ref, f1_ref):
        t0 = jnp.minimum(jnp.maximum(a0_ref[...], 0.0), 1.0) * float(K - 1)
        t1 = jnp.minimum(jnp.maximum(a1_ref[...], 0.0), 1.0) * float(K - 1)
        l0 = jnp.minimum(t0.astype(jnp.int32), K - 2)
        l1 = jnp.minimum(t1.astype(jnp.int32), K - 2)
        f0_ref[...] = t0 - l0.astype(jnp.float32)
        f1_ref[...] = t1 - l1.astype(jnp.float32)
        fl_ref[...] = s_ref[...] * KK + l0 * K + l1

    return pl.pallas_call(
        body,
        out_shape=[
            jax.ShapeDtypeStruct((_EB, 128), jnp.int32),
            jax.ShapeDtypeStruct((_EB, 128), jnp.float32),
            jax.ShapeDtypeStruct((_EB, 128), jnp.float32),
        ],
    )(src2, a02, a12)


def _dense1(x, w1f, r1):
    def body(x_ref, w_ref, r_ref, o1_ref, o2_ref):
        xv = x_ref[...]
        o1_ref[...] = _dot(xv, w_ref[...])
        o2_ref[...] = _dot(xv, r_ref[...])

    return pl.pallas_call(
        body,
        grid=(_NB,),
        in_specs=[
            pl.BlockSpec((_RB, DF), lambda i: (i, 0)),
            pl.BlockSpec((DF, KK * H1), lambda i: (0, 0)),
            pl.BlockSpec((DF, H1), lambda i: (0, 0)),
        ],
        out_specs=[
            pl.BlockSpec((_RB, KK * H1), lambda i: (i, 0)),
            pl.BlockSpec((_RB, H1), lambda i: (i, 0)),
        ],
        out_shape=[
            jax.ShapeDtypeStruct((N, KK * H1), jnp.float32),
            jax.ShapeDtypeStruct((N, H1), jnp.float32),
        ],
    )(x, w1f, r1)


def _dense2(p0, p1, d0, d1, root1, b1, w2f, r2, b2):
    def body(p0_ref, p1_ref, d0_ref, d1_ref, rt_ref, b1_ref, w_ref, r2_ref,
             b2_ref, t2_ref, ro_ref, iv_ref):
        deg = d0_ref[...][:, 0:1] + d1_ref[...][:, 0:1]
        inv = 1.0 / jnp.maximum(deg, 1.0)
        h = (p0_ref[...] + p1_ref[...]) * inv + rt_ref[...] + b1_ref[...]
        h = jnp.maximum(h, 0.0)
        t2_ref[...] = _dot(h, w_ref[...])
        ro_ref[...] = _dot(h, r2_ref[...]) + b2_ref[...]
        iv_ref[...] = inv

    return pl.pallas_call(
        body,
        grid=(_NB,),
        in_specs=[
            pl.BlockSpec((_RB, H1), lambda i: (i, 0)),
            pl.BlockSpec((_RB, H1), lambda i: (i, 0)),
            pl.BlockSpec((_RB, 16), lambda i: (i, 0)),
            pl.BlockSpec((_RB, 16), lambda i: (i, 0)),
            pl.BlockSpec((_RB, H1), lambda i: (i, 0)),
            pl.BlockSpec((1, H1), lambda i: (0, 0)),
            pl.BlockSpec((H1, KK * H2), lambda i: (0, 0)),
            pl.BlockSpec((H1, H2), lambda i: (0, 0)),
            pl.BlockSpec((1, H2), lambda i: (0, 0)),
        ],
        out_specs=[
            pl.BlockSpec((_RB, KK * H2), lambda i: (i, 0)),
            pl.BlockSpec((_RB, H2), lambda i: (i, 0)),
            pl.BlockSpec((_RB, 1), lambda i: (i, 0)),
        ],
        out_shape=[
            jax.ShapeDtypeStruct((N, KK * H2), jnp.float32),
            jax.ShapeDtypeStruct((N, H2), jnp.float32),
            jax.ShapeDtypeStruct((N, 1), jnp.float32),
        ],
    )(p0, p1, d0, d1, root1, b1, w2f, r2, b2)


def _head(q0, q1, inv, root2, batch2d, lw1, lb1, lw2, lb2, lw3, lb3):
    def body(q0_ref, q1_ref, iv_ref, rt_ref, b_ref, w1_ref, c1_ref, w2_ref,
             c2_ref, w3_ref, c3_ref, o_ref):
        h = (q0_ref[...] + q1_ref[...]) * iv_ref[...] + rt_ref[...]
        h = jnp.maximum(h, 0.0)
        gid = lax.broadcasted_iota(jnp.int32, (G, N), 0)
        ohm = (gid == b_ref[...]).astype(jnp.float32)
        s = _dot(ohm, h)
        cnt = jnp.sum(ohm, axis=1, keepdims=True)
        g = s / jnp.maximum(cnt, 1.0)
        z1 = _dot(g, w1_ref[...]) + c1_ref[...]
        g = g * (1.0 / (1.0 + jnp.exp(-z1)))
        z2 = _dot(g, w2_ref[...]) + c2_ref[...]
        g = g * (1.0 / (1.0 + jnp.exp(-z2)))
        z = _dot(g, w3_ref[...]) + c3_ref[...]
        zz = z - jnp.max(z, axis=1, keepdims=True)
        o_ref[...] = zz - jnp.log(jnp.sum(jnp.exp(zz), axis=1, keepdims=True))

    return pl.pallas_call(
        body,
        out_shape=jax.ShapeDtypeStruct((G, NCLS), jnp.float32),
    )(q0, q1, inv, root2, batch2d, lw1, lb1, lw2, lb2, lw3, lb3)


_edge32 = _make_edge_pass(H1, with_deg=True)
_edge64 = _make_edge_pass(H2, with_deg=False)


def kernel(x, edge_index, edge_attr, batch, W1, R1, b1, W2, R2, b2,
           lw1, lb1, lw2, lb2, lw3, lb3):
    src = edge_index[0]
    dst = edge_index[1]
    a0 = edge_attr[:, 0]
    a1 = edge_attr[:, 1]
    w1f = jnp.transpose(W1, (1, 0, 2)).reshape(DF, KK * H1)
    w2f = jnp.transpose(W2, (1, 0, 2)).reshape(H1, KK * H2)

    fl2, f02, f12 = _edge_prep(src.reshape(_EB, 128), a0.reshape(_EB, 128),
                               a1.reshape(_EB, 128))
    fl = fl2.reshape(E)
    f0b = jnp.broadcast_to(f02.reshape(E, 1), (E, 16))
    f1b = jnp.broadcast_to(f12.reshape(E, 1), (E, 16))

    xw1, root1 = _dense1(x, w1f, R1)
    tab1 = xw1.reshape(N * KK, H1)

    z32 = jnp.zeros((NRD, H1), jnp.float32)
    z16 = jnp.zeros((NRD, 16), jnp.float32)
    z64 = jnp.zeros((NRD, H2), jnp.float32)
    ones = (lax.broadcasted_iota(jnp.int32, (CH, 16), 1) == 0).astype(
        jnp.float32)

    p1, dg = _edge32(fl, dst, f0b, f1b, tab1, z32, z16, ones)

    t2, root2, inv = _dense2(p1[0], p1[1], dg[0], dg[1], root1,
                             b1.reshape(1, H1), w2f, R2, b2.reshape(1, H2))
    tab2 = t2.reshape(N * KK, H2)

    p2 = _edge64(fl, dst, f0b, f1b, tab2, z64)
    if isinstance(p2, (list, tuple)):
        p2 = p2[0]

    return _head(p2[0], p2[1], inv, root2, batch.reshape(1, N),
                 lw1, lb1.reshape(1, H2), lw2, lb2.reshape(1, H2),
                 lw3, lb3.reshape(1, NCLS))


# speedup vs baseline: 1.1654x; 1.1654x over previous
import jax
import jax.numpy as jnp
from jax import lax
from jax.experimental import pallas as pl
from jax.experimental.pallas import tpu as pltpu
from jax.experimental.pallas import tpu_sc as plsc

N = 10000
E = 320000
K = 5
KK = K * K
DF = 128
H1 = 32
H2 = 64
G = 64
NCLS = 10

CH = 80
NW = 32
CPW = E // (NW * CH)
NRD = 200

_HIGH = jax.lax.Precision.HIGHEST
_SC_PARAMS = pltpu.CompilerParams(use_tc_tiling_on_sc=False)


def _dot(a, b):
    return jnp.dot(a, b, preferred_element_type=jnp.float32, precision=_HIGH)


def _make_edge_pass(width, with_deg):
    mesh = plsc.VectorSubcoreMesh(core_axis_name="c", subcore_axis_name="s")
    outs = [jax.ShapeDtypeStruct((2, N, width), jnp.float32)]
    if with_deg:
        outs.append(jax.ShapeDtypeStruct((2, N, 16), jnp.float32))
    scratch = [
        pltpu.VMEM((2 * CH,), jnp.int32),
        pltpu.VMEM((2, CH), jnp.int32),
        pltpu.VMEM((2 * CH, 16), jnp.float32),
        pltpu.VMEM((2 * CH, 16), jnp.float32),
        pltpu.VMEM((2, CH), jnp.int32),
        pltpu.VMEM((2, CH), jnp.int32),
        pltpu.VMEM((2, CH), jnp.int32),
        pltpu.VMEM((2, CH), jnp.int32),
        pltpu.VMEM((8 * CH, width), jnp.float32),
        pltpu.VMEM((2 * CH, width), jnp.float32),
        pltpu.VMEM((2, CH), jnp.int32),
        pltpu.VMEM_SHARED((N, width), jnp.float32),
        pltpu.SemaphoreType.DMA,
        pltpu.SemaphoreType.DMA,
        pltpu.SemaphoreType.DMA,
    ]
    if with_deg:
        scratch += [
            pltpu.VMEM((CH, 16), jnp.float32),
            pltpu.VMEM((NRD, 16), jnp.float32),
            pltpu.VMEM_SHARED((N, 16), jnp.float32),
        ]

    def body(*refs):
        if with_deg:
            (fl_h, dst_h, f0_h, f1_h, tab_h, zv_h, zd_h, ones_h,
             out_h, deg_h,
             fl_v, dst_v, f0_v, f1_v, i0_v, i1_v, i2_v, i3_v, rows_v, msg_v,
             dsh_v, acc, sem_i, sem_g, sem_s, ones_v, dbuf_v, dacc) = refs
        else:
            (fl_h, dst_h, f0_h, f1_h, tab_h, zv_h,
             out_h,
             fl_v, dst_v, f0_v, f1_v, i0_v, i1_v, i2_v, i3_v, rows_v, msg_v,
             dsh_v, acc, sem_i, sem_g, sem_s) = refs

        cid = lax.axis_index("c")
        sid = lax.axis_index("s")
        w = cid * 16 + sid
        ebase = w * CPW * CH

        def in_copies(j, b):
            e0 = ebase + j * CH
            return [
                pltpu.make_async_copy(fl_h.at[pl.ds(e0, CH)],
                                      fl_v.at[pl.ds(b * CH, CH)], sem_i),
                pltpu.make_async_copy(dst_h.at[pl.ds(e0, CH)],
                                      dst_v.at[b], sem_i),
                pltpu.make_async_copy(f0_h.at[pl.ds(e0, CH)],
                                      f0_v.at[pl.ds(b * CH, CH)], sem_i),
                pltpu.make_async_copy(f1_h.at[pl.ds(e0, CH)],
                                      f1_v.at[pl.ds(b * CH, CH)], sem_i),
            ]

        def gather_copies(b):
            return [
                pltpu.make_async_copy(tab_h.at[iv.at[b]],
                                      rows_v.at[pl.ds((4 * b + c) * CH, CH)],
                                      sem_g)
                for c, iv in enumerate((i0_v, i1_v, i2_v, i3_v))
            ]

        def scatter_copies(b):
            cps = [pltpu.make_async_copy(
                msg_v.at[pl.ds(b * CH, CH)], acc.at[dsh_v.at[b]], sem_s)]
            if with_deg:
                cps.append(pltpu.make_async_copy(
                    ones_v, dacc.at[dsh_v.at[b]], sem_s))
            return cps

        def shadow_dst(b):
            def grp(m, c2):
                dsh_v[b, pl.ds(m * 16, 16)] = dst_v[b, pl.ds(m * 16, 16)]
                return c2
            lax.fori_loop(0, CH // 16, grp, 0)

        def comp_idx(b):
            def grp(m, c2):
                o = b * CH + m * 16
                fl = fl_v[pl.ds(o, 16)]
                i0_v[b, pl.ds(m * 16, 16)] = fl
                i1_v[b, pl.ds(m * 16, 16)] = fl + 1
                i2_v[b, pl.ds(m * 16, 16)] = fl + K
                i3_v[b, pl.ds(m * 16, 16)] = fl + (K + 1)
                return c2
            lax.fori_loop(0, CH // 16, grp, 0)

        def weight(b):
            rb = 4 * b * CH

            def edge(i2, c2):
                for u in range(2):
                    i = i2 * 2 + u
                    f0 = f0_v[b * CH + i, pl.ds(0, 16)]
                    f1 = f1_v[b * CH + i, pl.ds(0, 16)]
                    for kk in range(width // 16):
                        o = kk * 16
                        r00 = rows_v[rb + i, pl.ds(o, 16)]
                        r01 = rows_v[rb + CH + i, pl.ds(o, 16)]
                        r10 = rows_v[rb + 2 * CH + i, pl.ds(o, 16)]
                        r11 = rows_v[rb + 3 * CH + i, pl.ds(o, 16)]
                        a = r00 + f1 * (r01 - r00)
                        bq = r10 + f1 * (r11 - r10)
                        msg_v[b * CH + i, pl.ds(o, 16)] = a + f0 * (bq - a)
                return c2

            lax.fori_loop(0, CH // 2, edge, 0)

        bounce = rows_v.at[pl.ds(0, NRD)]

        @pl.when(sid < 10)
        def _init():
            pltpu.sync_copy(zv_h, bounce)
            for p in range(5):
                pltpu.sync_copy(
                    bounce, acc.at[pl.ds(sid * 1000 + p * NRD, NRD)])
            if with_deg:
                pltpu.sync_copy(zd_h, dbuf_v)
                for p in range(5):
                    pltpu.sync_copy(
                        dbuf_v, dacc.at[pl.ds(sid * 1000 + p * NRD, NRD)])
        if with_deg:
            pltpu.sync_copy(ones_h, ones_v)
        plsc.subcore_barrier()

        for cp in in_copies(0, 0):
            cp.start()
        for cp in in_copies(0, 0):
            cp.wait()
        comp_idx(0)
        for cp in gather_copies(0):
            cp.start()

        def do_chunk(j, b, first, last):
            nb = 1 - b

            if not first:
                for cp in scatter_copies(nb):
                    cp.wait()

            if not last:
                for cp in in_copies(j + 1, nb):
                    cp.start()

            for cp in gather_copies(b):
                cp.wait()

            if not last:
                for cp in in_copies(j + 1, nb):
                    cp.wait()
                comp_idx(nb)
                for cp in gather_copies(nb):
                    cp.start()

            weight(b)
            shadow_dst(b)

            for cp in scatter_copies(b):
                cp.start(add=True)

        do_chunk(0, 0, first=True, last=False)

        def pair_body(t, carry):
            j = 1 + t * 2
            do_chunk(j, 1, first=False, last=False)
            do_chunk(j + 1, 0, first=False, last=False)
            return carry

        lax.fori_loop(0, (CPW - 3) // 2, pair_body, 0)
        do_chunk(CPW - 2, 1, first=False, last=False)
        do_chunk(CPW - 1, 0, first=False, last=True)
        for cp in scatter_copies((CPW - 1) % 2):
            cp.wait()
        plsc.subcore_barrier()

        @pl.when(sid < 10)
        def _readout():
            for p in range(5):
                o = sid * 1000 + p * NRD
                pltpu.sync_copy(acc.at[pl.ds(o, NRD)], bounce)
                pltpu.sync_copy(bounce, out_h.at[cid, pl.ds(o, NRD)])
            if with_deg:
                for p in range(5):
                    o = sid * 1000 + p * NRD
                    pltpu.sync_copy(dacc.at[pl.ds(o, NRD)], dbuf_v)
                    pltpu.sync_copy(dbuf_v, deg_h.at[cid, pl.ds(o, NRD)])

    return pl.kernel(body, out_type=outs, mesh=mesh, scratch_types=scratch,
                     compiler_params=_SC_PARAMS)


_EB = 2500
_RB = 1000
_NB = N // _RB


def _edge_prep(src2, a02, a12):
    def body(s_ref, a0_ref, a1_ref, fl_ref, f0_ref, f1_ref):
        t0 = jnp.minimum(jnp.maximum(a0_ref[...], 0.0), 1.0) * float(K - 1)
        t1 = jnp.minimum(jnp.maximum(a1_ref[...], 0.0), 1.0) * float(K - 1)
        l0 = jnp.minimum(t0.astype(jnp.int32), K - 2)
        l1 = jnp.minimum(t1.astype(jnp.int32), K - 2)
        f0_ref[...] = t0 - l0.astype(jnp.float32)
        f1_ref[...] = t1 - l1.astype(jnp.float32)
        fl_ref[...] = s_ref[...] * KK + l0 * K + l1

    return pl.pallas_call(
        body,
        out_shape=[
            jax.ShapeDtypeStruct((_EB, 128), jnp.int32),
            jax.ShapeDtypeStruct((_EB, 128), jnp.float32),
            jax.ShapeDtypeStruct((_EB, 128), jnp.float32),
        ],
    )(src2, a02, a12)


def _dense1(x, w1f, r1):
    def body(x_ref, w_ref, r_ref, o1_ref, o2_ref):
        xv = x_ref[...]
        o1_ref[...] = _dot(xv, w_ref[...])
        o2_ref[...] = _dot(xv, r_ref[...])

    return pl.pallas_call(
        body,
        grid=(_NB,),
        in_specs=[
            pl.BlockSpec((_RB, DF), lambda i: (i, 0)),
            pl.BlockSpec((DF, KK * H1), lambda i: (0, 0)),
            pl.BlockSpec((DF, H1), lambda i: (0, 0)),
        ],
        out_specs=[
            pl.BlockSpec((_RB, KK * H1), lambda i: (i, 0)),
            pl.BlockSpec((_RB, H1), lambda i: (i, 0)),
        ],
        out_shape=[
            jax.ShapeDtypeStruct((N, KK * H1), jnp.float32),
            jax.ShapeDtypeStruct((N, H1), jnp.float32),
        ],
    )(x, w1f, r1)


def _dense2(p0, p1, d0, d1, root1, b1, w2f, r2, b2):
    def body(p0_ref, p1_ref, d0_ref, d1_ref, rt_ref, b1_ref, w_ref, r2_ref,
             b2_ref, t2_ref, ro_ref, iv_ref):
        deg = d0_ref[...][:, 0:1] + d1_ref[...][:, 0:1]
        inv = 1.0 / jnp.maximum(deg, 1.0)
        h = (p0_ref[...] + p1_ref[...]) * inv + rt_ref[...] + b1_ref[...]
        h = jnp.maximum(h, 0.0)
        t2_ref[...] = _dot(h, w_ref[...])
        ro_ref[...] = _dot(h, r2_ref[...]) + b2_ref[...]
        iv_ref[...] = inv

    return pl.pallas_call(
        body,
        grid=(_NB,),
        in_specs=[
            pl.BlockSpec((_RB, H1), lambda i: (i, 0)),
            pl.BlockSpec((_RB, H1), lambda i: (i, 0)),
            pl.BlockSpec((_RB, 16), lambda i: (i, 0)),
            pl.BlockSpec((_RB, 16), lambda i: (i, 0)),
            pl.BlockSpec((_RB, H1), lambda i: (i, 0)),
            pl.BlockSpec((1, H1), lambda i: (0, 0)),
            pl.BlockSpec((H1, KK * H2), lambda i: (0, 0)),
            pl.BlockSpec((H1, H2), lambda i: (0, 0)),
            pl.BlockSpec((1, H2), lambda i: (0, 0)),
        ],
        out_specs=[
            pl.BlockSpec((_RB, KK * H2), lambda i: (i, 0)),
            pl.BlockSpec((_RB, H2), lambda i: (i, 0)),
            pl.BlockSpec((_RB, 1), lambda i: (i, 0)),
        ],
        out_shape=[
            jax.ShapeDtypeStruct((N, KK * H2), jnp.float32),
            jax.ShapeDtypeStruct((N, H2), jnp.float32),
            jax.ShapeDtypeStruct((N, 1), jnp.float32),
        ],
    )(p0, p1, d0, d1, root1, b1, w2f, r2, b2)


def _head(q0, q1, inv, root2, batch2d, lw1, lb1, lw2, lb2, lw3, lb3):
    def body(q0_ref, q1_ref, iv_ref, rt_ref, b_ref, w1_ref, c1_ref, w2_ref,
             c2_ref, w3_ref, c3_ref, o_ref):
        h = (q0_ref[...] + q1_ref[...]) * iv_ref[...] + rt_ref[...]
        h = jnp.maximum(h, 0.0)
        gid = lax.broadcasted_iota(jnp.int32, (G, N), 0)
        ohm = (gid == b_ref[...]).astype(jnp.float32)
        s = _dot(ohm, h)
        cnt = jnp.sum(ohm, axis=1, keepdims=True)
        g = s / jnp.maximum(cnt, 1.0)
        z1 = _dot(g, w1_ref[...]) + c1_ref[...]
        g = g * (1.0 / (1.0 + jnp.exp(-z1)))
        z2 = _dot(g, w2_ref[...]) + c2_ref[...]
        g = g * (1.0 / (1.0 + jnp.exp(-z2)))
        z = _dot(g, w3_ref[...]) + c3_ref[...]
        zz = z - jnp.max(z, axis=1, keepdims=True)
        o_ref[...] = zz - jnp.log(jnp.sum(jnp.exp(zz), axis=1, keepdims=True))

    return pl.pallas_call(
        body,
        out_shape=jax.ShapeDtypeStruct((G, NCLS), jnp.float32),
    )(q0, q1, inv, root2, batch2d, lw1, lb1, lw2, lb2, lw3, lb3)


_edge32 = _make_edge_pass(H1, with_deg=True)
_edge64 = _make_edge_pass(H2, with_deg=False)


def kernel(x, edge_index, edge_attr, batch, W1, R1, b1, W2, R2, b2,
           lw1, lb1, lw2, lb2, lw3, lb3):
    src = edge_index[0]
    dst = edge_index[1]
    a0 = edge_attr[:, 0]
    a1 = edge_attr[:, 1]
    w1f = jnp.transpose(W1, (1, 0, 2)).reshape(DF, KK * H1)
    w2f = jnp.transpose(W2, (1, 0, 2)).reshape(H1, KK * H2)

    fl2, f02, f12 = _edge_prep(src.reshape(_EB, 128), a0.reshape(_EB, 128),
                               a1.reshape(_EB, 128))
    fl = fl2.reshape(E)
    f0b = jnp.broadcast_to(f02.reshape(E, 1), (E, 16))
    f1b = jnp.broadcast_to(f12.reshape(E, 1), (E, 16))

    xw1, root1 = _dense1(x, w1f, R1)
    tab1 = xw1.reshape(N * KK, H1)

    z32 = jnp.zeros((NRD, H1), jnp.float32)
    z16 = jnp.zeros((NRD, 16), jnp.float32)
    z64 = jnp.zeros((NRD, H2), jnp.float32)
    ones = (lax.broadcasted_iota(jnp.int32, (CH, 16), 1) == 0).astype(
        jnp.float32)

    p1, dg = _edge32(fl, dst, f0b, f1b, tab1, z32, z16, ones)

    t2, root2, inv = _dense2(p1[0], p1[1], dg[0], dg[1], root1,
                             b1.reshape(1, H1), w2f, R2, b2.reshape(1, H2))
    tab2 = t2.reshape(N * KK, H2)

    p2 = _edge64(fl, dst, f0b, f1b, tab2, z64)
    if isinstance(p2, (list, tuple)):
        p2 = p2[0]

    return _head(p2[0], p2[1], inv, root2, batch.reshape(1, N),
                 lw1, lb1.reshape(1, H2), lw2, lb2.reshape(1, H2),
                 lw3, lb3.reshape(1, NCLS))
